# baseline (speedup 1.0000x reference)
import jax
import jax.numpy as jnp
from jax.experimental import pallas as pl

_NUM_RAYS = 1024
_NUM_LIGHTS = 32
_BATCH = 4096
_NB = 512


def _gather_normed(idx_ref, pt_ref):
    idxv = idx_ref[...]
    lanes = jax.lax.broadcasted_iota(jnp.int32, (_NB, _NUM_LIGHTS), 1)
    oh = (lanes == idxv).astype(jnp.float32)
    pt = pt_ref[...]
    x = jnp.sum(oh * pt[0:1, :], axis=1, keepdims=True)
    y = jnp.sum(oh * pt[1:2, :], axis=1, keepdims=True)
    z = -jnp.abs(jnp.sum(oh * pt[2:3, :], axis=1, keepdims=True))
    inten = jnp.abs(jnp.sum(oh * pt[3:4, :], axis=1, keepdims=True))
    n = jnp.sqrt(x * x + y * y + z * z)
    inv = 1.0 / jnp.maximum(n, 1e-12)
    return x * inv, y * inv, z * inv, inten


def _ld_kernel(idx_ref, pt_ref, ld_ref):
    xn, yn, zn, _ = _gather_normed(idx_ref, pt_ref)
    shape = (_NB, _NUM_RAYS)
    ld_ref[0] = jnp.broadcast_to(xn, shape)
    ld_ref[1] = jnp.broadcast_to(yn, shape)
    ld_ref[2] = jnp.broadcast_to(zn, shape)


def _li_kernel(idx_ref, pt_ref, li_ref):
    _, _, _, inten = _gather_normed(idx_ref, pt_ref)
    bi = jnp.broadcast_to(inten, (_NB, _NUM_RAYS))
    li_ref[0] = bi
    li_ref[1] = bi
    li_ref[2] = bi


def _run(body, idx2, params_t):
    grid = _BATCH // _NB
    return pl.pallas_call(
        body,
        grid=(grid,),
        in_specs=[
            pl.BlockSpec((_NB, 1), lambda i: (i, 0)),
            pl.BlockSpec((4, _NUM_LIGHTS), lambda i: (0, 0)),
        ],
        out_specs=[pl.BlockSpec((3, _NB, _NUM_RAYS), lambda i: (0, i, 0))],
        out_shape=[jax.ShapeDtypeStruct((3, _BATCH, _NUM_RAYS), jnp.float32)],
    )(idx2, params_t)[0]


def kernel(idx, light_direction_xy, light_direction_z, light_intensity):
    params_t = jnp.concatenate(
        [light_direction_xy, light_direction_z, light_intensity], axis=1
    ).T
    idx2 = idx.reshape(_BATCH, 1)

    p_ld = _run(_ld_kernel, idx2, params_t)
    p_li = _run(_li_kernel, idx2, params_t)
    out_ld = p_ld.transpose(1, 2, 0).reshape(-1, 3)
    out_li = p_li.transpose(1, 2, 0).reshape(-1, 3)
    return (out_ld, out_li)


# speedup vs baseline: 1.0372x; 1.0372x over previous
import jax
import jax.numpy as jnp
from jax.experimental import pallas as pl

_NUM_RAYS = 1024
_NUM_LIGHTS = 32
_BATCH = 4096
_NB = 1024


def _light_kernel(idx_ref, pt_ref, ld_ref, li_ref):
    idxv = idx_ref[...]
    lanes = jax.lax.broadcasted_iota(jnp.int32, (_NB, _NUM_LIGHTS), 1)
    oh = (lanes == idxv).astype(jnp.float32)
    pt = pt_ref[...]
    x = jnp.sum(oh * pt[0:1, :], axis=1, keepdims=True)
    y = jnp.sum(oh * pt[1:2, :], axis=1, keepdims=True)
    z = -jnp.abs(jnp.sum(oh * pt[2:3, :], axis=1, keepdims=True))
    inten = jnp.abs(jnp.sum(oh * pt[3:4, :], axis=1, keepdims=True))
    n = jnp.sqrt(x * x + y * y + z * z)
    inv = 1.0 / jnp.maximum(n, 1e-12)
    shape = (_NB, _NUM_RAYS)
    ld_ref[0] = jnp.broadcast_to(x * inv, shape)
    ld_ref[1] = jnp.broadcast_to(y * inv, shape)
    ld_ref[2] = jnp.broadcast_to(z * inv, shape)
    li_ref[...] = jnp.broadcast_to(inten, shape)


def kernel(idx, light_direction_xy, light_direction_z, light_intensity):
    params_t = jnp.concatenate(
        [light_direction_xy, light_direction_z, light_intensity], axis=1
    ).T
    idx2 = idx.reshape(_BATCH, 1)
    grid = _BATCH // _NB

    p_ld, p_li = pl.pallas_call(
        _light_kernel,
        grid=(grid,),
        in_specs=[
            pl.BlockSpec((_NB, 1), lambda i: (i, 0)),
            pl.BlockSpec((4, _NUM_LIGHTS), lambda i: (0, 0)),
        ],
        out_specs=[
            pl.BlockSpec((3, _NB, _NUM_RAYS), lambda i: (0, i, 0)),
            pl.BlockSpec((_NB, _NUM_RAYS), lambda i: (i, 0)),
        ],
        out_shape=[
            jax.ShapeDtypeStruct((3, _BATCH, _NUM_RAYS), jnp.float32),
            jax.ShapeDtypeStruct((_BATCH, _NUM_RAYS), jnp.float32),
        ],
    )(idx2, params_t)

    out_ld = p_ld.transpose(1, 2, 0).reshape(-1, 3)
    out_li = jnp.broadcast_to(
        p_li.reshape(_BATCH * _NUM_RAYS, 1), (_BATCH * _NUM_RAYS, 3)
    )
    return (out_ld, out_li)


# speedup vs baseline: 1.0503x; 1.0127x over previous
import jax
import jax.numpy as jnp
from jax.experimental import pallas as pl

_NUM_RAYS = 1024
_NUM_LIGHTS = 32
_BATCH = 4096
_NB = 512


def _light_kernel(idx_ref, pt_ref, ld_ref, li_ref):
    idxv = idx_ref[...]
    lanes = jax.lax.broadcasted_iota(jnp.int32, (_NB, _NUM_LIGHTS), 1)
    oh = (lanes == idxv).astype(jnp.float32)
    pt = pt_ref[...]
    x = jnp.sum(oh * pt[0:1, :], axis=1, keepdims=True)
    y = jnp.sum(oh * pt[1:2, :], axis=1, keepdims=True)
    z = -jnp.abs(jnp.sum(oh * pt[2:3, :], axis=1, keepdims=True))
    inten = jnp.abs(jnp.sum(oh * pt[3:4, :], axis=1, keepdims=True))
    n = jnp.sqrt(x * x + y * y + z * z)
    inv = 1.0 / jnp.maximum(n, 1e-12)
    shape = (_NB, _NUM_RAYS)
    ld_ref[0] = jnp.broadcast_to(x * inv, shape)
    ld_ref[1] = jnp.broadcast_to(y * inv, shape)
    ld_ref[2] = jnp.broadcast_to(z * inv, shape)
    li_ref[...] = jnp.broadcast_to(inten, shape)


def kernel(idx, light_direction_xy, light_direction_z, light_intensity):
    params_t = jnp.concatenate(
        [light_direction_xy, light_direction_z, light_intensity], axis=1
    ).T
    idx2 = idx.reshape(_BATCH, 1)
    grid = _BATCH // _NB

    p_ld, p_li = pl.pallas_call(
        _light_kernel,
        grid=(grid,),
        in_specs=[
            pl.BlockSpec((_NB, 1), lambda i: (i, 0)),
            pl.BlockSpec((4, _NUM_LIGHTS), lambda i: (0, 0)),
        ],
        out_specs=[
            pl.BlockSpec((3, _NB, _NUM_RAYS), lambda i: (0, i, 0)),
            pl.BlockSpec((_NB, _NUM_RAYS), lambda i: (i, 0)),
        ],
        out_shape=[
            jax.ShapeDtypeStruct((3, _BATCH, _NUM_RAYS), jnp.float32),
            jax.ShapeDtypeStruct((_BATCH, _NUM_RAYS), jnp.float32),
        ],
    )(idx2, params_t)

    out_ld = p_ld.transpose(1, 2, 0).reshape(-1, 3)
    out_li = jnp.broadcast_to(
        p_li.reshape(_BATCH * _NUM_RAYS, 1), (_BATCH * _NUM_RAYS, 3)
    )
    return (out_ld, out_li)
